# baseline (speedup 1.0000x reference)
import jax
import jax.numpy as jnp
from jax import lax
from jax.experimental import pallas as pl

B = 4
C = 256
H8 = 32
W8 = 64
Q = H8 * W8
R = 4

_OFFS = []
for _y in range(-R, R + 1):
    for _x in range(abs(_y) - R, R - abs(_y) + 1):
        _OFFS.append((_x, _y))
L = len(_OFFS)

_XPAR = [((float((1 << k) * ((W8 >> k) - 1)) / 512.0), W8 >> k) for k in range(4)]
_YPAR = [((float((1 << k) * ((H8 >> k) - 1)) / 256.0), H8 >> k) for k in range(4)]


def _all_weight_vectors(pbs):
    offs = (lax.broadcasted_iota(jnp.int32, (9, 1), 0) - R).astype(jnp.float32)
    prows = []
    ws_list = []
    nmax_rows = []
    for pb, s, n in pbs:
        prows.append(pb[None, :] + offs * s)
        f = jnp.floor(pb - 4.0 * s)
        ws_list.append(jnp.clip(f, 0.0, float(n - 3)).astype(jnp.int32))
        nmax_rows.append(jnp.full((9, Q), float(n - 1), jnp.float32))
    P = jnp.concatenate(prows, axis=0)
    NM = jnp.concatenate(nmax_rows, axis=0)
    WS = jnp.concatenate(
        [jnp.broadcast_to(w[None, :], (9, Q)) for w in ws_list], axis=0)
    p0 = jnp.floor(P)
    w1 = P - p0
    m0 = (1.0 - w1) * ((p0 >= 0.0) & (p0 <= NM)).astype(jnp.float32)
    m1 = w1 * ((p0 + 1.0 >= 0.0) & (p0 + 1.0 <= NM)).astype(jnp.float32)
    d0 = p0.astype(jnp.int32) - WS
    w3 = [jnp.where(d0 == d, m0, 0.0) + jnp.where(d0 == d - 1, m1, 0.0)
          for d in range(3)]
    W3 = jnp.stack(w3, axis=1)
    return ws_list, [W3[9 * i:9 * (i + 1)] for i in range(8)]


def _lookup_body(f2t_ref, f1_ref, flow_ref, out_ref):
    fy = flow_ref[0]
    fx = flow_ref[1]

    qi = lax.broadcasted_iota(jnp.int32, (Q,), 0)
    jj = (qi % W8).astype(jnp.float32)
    ii = (qi // W8).astype(jnp.float32)

    pbs = ([((jj + fx) * (float((W8 >> k) - 1) / 512.0),) + _XPAR[k]
            for k in range(4)]
           + [((ii + fy) * (float((H8 >> k) - 1) / 256.0),) + _YPAR[k]
              for k in range(4)])
    starts, w3s = _all_weight_vectors(pbs)

    sel = []
    for k in range(4):
        hk, wk = H8 >> k, W8 >> k
        xs, ys = starts[k], starts[4 + k]
        ysh = ys >> 1
        ymasks = [(ysh == m).astype(jnp.bfloat16)[None, :]
                  for m in range(hk // 2 - 1)]
        ypar = (ys & 1).astype(jnp.bfloat16)[None, :]
        nmx = wk // 2
        xsh = (xs >> 1)[None, :]
        mi = lax.broadcasted_iota(jnp.int32, (nmx, Q), 0)
        xm0 = (mi == xsh).astype(jnp.bfloat16)
        xm1 = (mi == xsh + 1).astype(jnp.bfloat16)
        xpar = (xs & 1).astype(jnp.bfloat16)[None, :]
        sel.append((ymasks, ypar, 1.0 - ypar, xm0, xm1, xpar, 1.0 - xpar))

    for b in range(B):
        _one_batch(f2t_ref[b], f1_ref[b], sel, w3s, out_ref, b)


def _one_batch(f2t, f1, sel, w3s, out_ref, b):
    corrT = (jnp.dot(f2t, f1, preferred_element_type=jnp.float32)
             * (1.0 / 16.0)).astype(jnp.bfloat16)

    vol = corrT.reshape(H8, W8, Q)
    outs = []
    for k in range(4):
        hk = H8 >> k
        wk = W8 >> k
        if k > 0:
            a = vol.reshape(hk, 2, wk, 2, Q)
            vol = (a[:, 0, :, 0] + a[:, 0, :, 1] + a[:, 1, :, 0] + a[:, 1, :, 1]) * 0.25

        wx3, wy3 = w3s[k], w3s[4 + k]
        ymasks, ypar, nypar, xm0, xm1, xpar, nxpar = sel[k]

        E0 = jnp.zeros((wk, Q), jnp.bfloat16)
        O0 = jnp.zeros((wk, Q), jnp.bfloat16)
        E1 = jnp.zeros((wk, Q), jnp.bfloat16)
        O1 = jnp.zeros((wk, Q), jnp.bfloat16)
        for m in range(hk // 2 - 1):
            E0 = E0 + vol[2 * m] * ymasks[m]
            O0 = O0 + vol[2 * m + 1] * ymasks[m]
            E1 = E1 + vol[2 * m + 2] * ymasks[m]
            O1 = O1 + vol[2 * m + 3] * ymasks[m]
        rows = [E0 * nypar + O0 * ypar,
                O0 * nypar + E1 * ypar,
                E1 * nypar + O1 * ypar]

        win = []
        for dy in range(3):
            ar = rows[dy].reshape(wk // 2, 2, Q)
            ev, od = ar[:, 0], ar[:, 1]
            e0 = jnp.sum(ev * xm0, axis=0)
            o0 = jnp.sum(od * xm0, axis=0)
            e1 = jnp.sum(ev * xm1, axis=0)
            o1 = jnp.sum(od * xm1, axis=0)
            win.append([e0 * nxpar[0] + o0 * xpar[0],
                        o0 * nxpar[0] + e1 * xpar[0],
                        e1 * nxpar[0] + o1 * xpar[0]])

        winarr = jnp.stack([jnp.stack(w, axis=0) for w in win],
                           axis=0).astype(jnp.float32)
        t = jnp.sum(winarr[:, None, :, :] * wx3[None, :, :, :], axis=2)

        lvl = []
        for yo in range(-R, R + 1):
            a0, a1 = abs(yo), 9 - abs(yo)
            g = (wy3[yo + R, 0][None, :] * t[0, a0:a1]
                 + wy3[yo + R, 1][None, :] * t[1, a0:a1]
                 + wy3[yo + R, 2][None, :] * t[2, a0:a1])
            lvl.append(g)
        outs.append(jnp.concatenate(lvl, axis=0))

    out_ref[b] = jnp.stack(outs, axis=1)


@jax.jit
def kernel(feat1, feat2, curr_flow):
    f1 = feat1.reshape(B, C, Q).astype(jnp.bfloat16)
    f2t = feat2.reshape(B, C, Q).transpose(0, 2, 1).astype(jnp.bfloat16)
    flow = curr_flow.reshape(2, Q)

    out = pl.pallas_call(
        _lookup_body,
        out_shape=jax.ShapeDtypeStruct((B, L, 4, Q), jnp.float32),
    )(f2t, f1, flow)
    return out.reshape(B, L, 4, H8, W8)


# speedup vs baseline: 1.0770x; 1.0770x over previous
import jax
import jax.numpy as jnp
from jax import lax
from jax.experimental import pallas as pl

B = 4
C = 256
H8 = 32
W8 = 64
Q = H8 * W8
R = 4

_OFFS = []
for _y in range(-R, R + 1):
    for _x in range(abs(_y) - R, R - abs(_y) + 1):
        _OFFS.append((_x, _y))
L = len(_OFFS)

_XPAR = [((float((1 << k) * ((W8 >> k) - 1)) / 512.0), W8 >> k) for k in range(4)]
_YPAR = [((float((1 << k) * ((H8 >> k) - 1)) / 256.0), H8 >> k) for k in range(4)]


def _all_weight_vectors(pbs):
    offs = (lax.broadcasted_iota(jnp.int32, (9, 1), 0) - R).astype(jnp.float32)
    prows = []
    ws_list = []
    nmax_rows = []
    for pb, s, n in pbs:
        prows.append(pb[None, :] + offs * s)
        f = jnp.floor(pb - 4.0 * s)
        ws_list.append(jnp.clip(f, 0.0, float(n - 3)).astype(jnp.int32))
        nmax_rows.append(jnp.full((9, Q), float(n - 1), jnp.float32))
    P = jnp.concatenate(prows, axis=0)
    NM = jnp.concatenate(nmax_rows, axis=0)
    WS = jnp.concatenate(
        [jnp.broadcast_to(w[None, :], (9, Q)) for w in ws_list], axis=0)
    p0 = jnp.floor(P)
    w1 = P - p0
    m0 = (1.0 - w1) * ((p0 >= 0.0) & (p0 <= NM)).astype(jnp.float32)
    m1 = w1 * ((p0 + 1.0 >= 0.0) & (p0 + 1.0 <= NM)).astype(jnp.float32)
    d0 = p0.astype(jnp.int32) - WS
    w3 = [jnp.where(d0 == d, m0, 0.0) + jnp.where(d0 == d - 1, m1, 0.0)
          for d in range(3)]
    W3 = jnp.stack(w3, axis=1)
    return ws_list, [W3[9 * i:9 * (i + 1)] for i in range(8)]


def _lookup_body(f2t_ref, f1_ref, flow_ref, out_ref):
    fy = flow_ref[0]
    fx = flow_ref[1]

    qi = lax.broadcasted_iota(jnp.int32, (Q,), 0)
    jj = (qi % W8).astype(jnp.float32)
    ii = (qi // W8).astype(jnp.float32)

    pbs = ([((jj + fx) * (float((W8 >> k) - 1) / 512.0),) + _XPAR[k]
            for k in range(4)]
           + [((ii + fy) * (float((H8 >> k) - 1) / 256.0),) + _YPAR[k]
              for k in range(4)])
    starts, w3s = _all_weight_vectors(pbs)

    for b in range(B):
        _one_batch(f2t_ref[b], f1_ref[b], starts, w3s, out_ref, b)


def _one_batch(f2t, f1, starts, w3s, out_ref, b):
    corrT = (jnp.dot(f2t, f1, preferred_element_type=jnp.float32)
             * (1.0 / 16.0)).astype(jnp.bfloat16)

    vol = corrT.reshape(H8, W8, Q)
    outs = []
    for k in range(4):
        hk = H8 >> k
        wk = W8 >> k
        if k > 0:
            a = vol.reshape(hk, 2, wk, 2, Q)
            vol = (a[:, 0, :, 0] + a[:, 0, :, 1] + a[:, 1, :, 0] + a[:, 1, :, 1]) * 0.25

        xs, wx3 = starts[k], w3s[k]
        ys, wy3 = starts[4 + k], w3s[4 + k]

        ysh = ys >> 1
        par = (ys & 1).astype(jnp.bfloat16)[None, :]
        npar = 1.0 - par
        nm = hk // 2
        masks = [(ysh == m).astype(jnp.bfloat16)[None, :] for m in range(nm - 1)]
        E0 = jnp.zeros((wk, Q), jnp.bfloat16)
        O0 = jnp.zeros((wk, Q), jnp.bfloat16)
        E1 = jnp.zeros((wk, Q), jnp.bfloat16)
        O1 = jnp.zeros((wk, Q), jnp.bfloat16)
        for m in range(nm - 1):
            E0 = E0 + vol[2 * m] * masks[m]
            O0 = O0 + vol[2 * m + 1] * masks[m]
            E1 = E1 + vol[2 * m + 2] * masks[m]
            O1 = O1 + vol[2 * m + 3] * masks[m]
        rows = [E0 * npar + O0 * par,
                O0 * npar + E1 * par,
                E1 * npar + O1 * par]

        lxi = lax.broadcasted_iota(jnp.int32, (wk, Q), 0)
        xmasks = [(lxi == (xs + dx)[None, :]).astype(jnp.bfloat16)
                  for dx in range(3)]
        win = [[jnp.sum(rows[dy] * xmasks[dx], axis=0) for dx in range(3)]
               for dy in range(3)]

        winarr = jnp.stack([jnp.stack(w, axis=0) for w in win],
                           axis=0).astype(jnp.float32)
        t = jnp.sum(winarr[:, None, :, :] * wx3[None, :, :, :], axis=2)

        lvl = []
        for yo in range(-R, R + 1):
            a0, a1 = abs(yo), 9 - abs(yo)
            g = (wy3[yo + R, 0][None, :] * t[0, a0:a1]
                 + wy3[yo + R, 1][None, :] * t[1, a0:a1]
                 + wy3[yo + R, 2][None, :] * t[2, a0:a1])
            lvl.append(g)
        outs.append(jnp.concatenate(lvl, axis=0))

    out_ref[b] = jnp.stack(outs, axis=1)


@jax.jit
def kernel(feat1, feat2, curr_flow):
    f1 = feat1.reshape(B, C, Q).astype(jnp.bfloat16)
    f2t = feat2.reshape(B, C, Q).transpose(0, 2, 1).astype(jnp.bfloat16)
    flow = curr_flow.reshape(2, Q)

    out = pl.pallas_call(
        _lookup_body,
        out_shape=jax.ShapeDtypeStruct((B, L, 4, Q), jnp.float32),
    )(f2t, f1, flow)
    return out.reshape(B, L, 4, H8, W8)


# speedup vs baseline: 1.0774x; 1.0003x over previous
import jax
import jax.numpy as jnp
from jax import lax
from jax.experimental import pallas as pl

B = 4
C = 256
H8 = 32
W8 = 64
Q = H8 * W8
R = 4

_OFFS = []
for _y in range(-R, R + 1):
    for _x in range(abs(_y) - R, R - abs(_y) + 1):
        _OFFS.append((_x, _y))
L = len(_OFFS)

_XPAR = [((float((1 << k) * ((W8 >> k) - 1)) / 512.0), W8 >> k) for k in range(4)]
_YPAR = [((float((1 << k) * ((H8 >> k) - 1)) / 256.0), H8 >> k) for k in range(4)]


def _all_weight_vectors(pbs):
    offs = (lax.broadcasted_iota(jnp.int32, (9, 1), 0) - R).astype(jnp.float32)
    prows = []
    ws_list = []
    nmax_rows = []
    for pb, s, n in pbs:
        prows.append(pb[None, :] + offs * s)
        f = jnp.floor(pb - 4.0 * s)
        ws_list.append(jnp.clip(f, 0.0, float(n - 3)).astype(jnp.int32))
        nmax_rows.append(jnp.full((9, Q), float(n - 1), jnp.float32))
    P = jnp.concatenate(prows, axis=0)
    NM = jnp.concatenate(nmax_rows, axis=0)
    WS = jnp.concatenate(
        [jnp.broadcast_to(w[None, :], (9, Q)) for w in ws_list], axis=0)
    p0 = jnp.floor(P)
    w1 = P - p0
    m0 = (1.0 - w1) * ((p0 >= 0.0) & (p0 <= NM)).astype(jnp.float32)
    m1 = w1 * ((p0 + 1.0 >= 0.0) & (p0 + 1.0 <= NM)).astype(jnp.float32)
    d0 = p0.astype(jnp.int32) - WS
    w3 = [jnp.where(d0 == d, m0, 0.0) + jnp.where(d0 == d - 1, m1, 0.0)
          for d in range(3)]
    W3 = jnp.stack(w3, axis=1)
    return ws_list, [W3[9 * i:9 * (i + 1)] for i in range(8)]


def _lookup_body(f2t_ref, f1_ref, flow_ref, out_ref):
    fy = flow_ref[0]
    fx = flow_ref[1]

    qi = lax.broadcasted_iota(jnp.int32, (Q,), 0)
    jj = (qi % W8).astype(jnp.float32)
    ii = (qi // W8).astype(jnp.float32)

    pbs = ([((jj + fx) * (float((W8 >> k) - 1) / 512.0),) + _XPAR[k]
            for k in range(4)]
           + [((ii + fy) * (float((H8 >> k) - 1) / 256.0),) + _YPAR[k]
              for k in range(4)])
    starts, w3s = _all_weight_vectors(pbs)

    for b in range(B):
        _one_batch(f2t_ref[b], f1_ref[b], starts, w3s, out_ref, b)


def _one_batch(f2t, f1, starts, w3s, out_ref, b):
    corrT = (jnp.dot(f2t, f1, preferred_element_type=jnp.float32)
             * (1.0 / 16.0)).astype(jnp.bfloat16)

    vol = corrT.reshape(H8, W8, Q)
    outs = []
    for k in range(4):
        hk = H8 >> k
        wk = W8 >> k
        if k > 0:
            a = vol.reshape(hk, 2, wk, 2, Q)
            vol = (a[:, 0, :, 0] + a[:, 0, :, 1] + a[:, 1, :, 0] + a[:, 1, :, 1]) * 0.25

        xs, wx3 = starts[k], w3s[k]
        ys, wy3 = starts[4 + k], w3s[4 + k]

        acc = [jnp.zeros((wk, Q), jnp.bfloat16) for _ in range(3)]
        for y in range(hk):
            m = ((ys >= y - 2) & (ys <= y)).astype(jnp.bfloat16)[None, :]
            acc[y % 3] = acc[y % 3] + vol[y] * m
        rows = []
        for dy in range(3):
            rd = (ys + dy) % 3
            s = [(rd == r).astype(jnp.bfloat16)[None, :] for r in range(3)]
            rows.append(acc[0] * s[0] + acc[1] * s[1] + acc[2] * s[2])

        lxi = lax.broadcasted_iota(jnp.int32, (wk, Q), 0)
        xmasks = [(lxi == (xs + dx)[None, :]).astype(jnp.bfloat16)
                  for dx in range(3)]
        win = [[jnp.sum(rows[dy] * xmasks[dx], axis=0) for dx in range(3)]
               for dy in range(3)]

        winarr = jnp.stack([jnp.stack(w, axis=0) for w in win],
                           axis=0).astype(jnp.float32)
        t = jnp.sum(winarr[:, None, :, :] * wx3[None, :, :, :], axis=2)

        lvl = []
        for yo in range(-R, R + 1):
            a0, a1 = abs(yo), 9 - abs(yo)
            g = (wy3[yo + R, 0][None, :] * t[0, a0:a1]
                 + wy3[yo + R, 1][None, :] * t[1, a0:a1]
                 + wy3[yo + R, 2][None, :] * t[2, a0:a1])
            lvl.append(g)
        outs.append(jnp.concatenate(lvl, axis=0))

    out_ref[b] = jnp.stack(outs, axis=1)


@jax.jit
def kernel(feat1, feat2, curr_flow):
    f1 = feat1.reshape(B, C, Q).astype(jnp.bfloat16)
    f2t = feat2.reshape(B, C, Q).transpose(0, 2, 1).astype(jnp.bfloat16)
    flow = curr_flow.reshape(2, Q)

    out = pl.pallas_call(
        _lookup_body,
        out_shape=jax.ShapeDtypeStruct((B, L, 4, Q), jnp.float32),
    )(f2t, f1, flow)
    return out.reshape(B, L, 4, H8, W8)
